# baseline (speedup 1.0000x reference)
import functools

import jax
import jax.numpy as jnp
from jax import lax
from jax.experimental import pallas as pl
from jax.experimental.pallas import tpu as pltpu
from jax.experimental.pallas import tpu_sc as plsc

_NROWS = 16384
_NCOLS = 200
_W = _NCOLS // 4
_L = 16


def _sc_energy_shift(wpacked, energies, se_pad):
    info = plsc.get_sparse_core_info()
    nw = info.num_cores * info.num_subcores
    rows_w = _NROWS // nw
    groups = rows_w // _L

    mesh = plsc.VectorSubcoreMesh(core_axis_name="c", subcore_axis_name="s")

    @functools.partial(
        pl.kernel,
        mesh=mesh,
        out_type=jax.ShapeDtypeStruct((_NROWS,), jnp.float32),
        compiler_params=pltpu.CompilerParams(
            needs_layout_passes=False, use_tc_tiling_on_sc=False),
        scratch_types=[
            pltpu.VMEM((rows_w * _W,), jnp.int32),
            pltpu.VMEM((2048,), jnp.float32),
            pltpu.VMEM((_L,), jnp.float32),
            pltpu.VMEM((rows_w,), jnp.float32),
            pltpu.VMEM((rows_w,), jnp.float32),
        ],
    )
    def k(w_hbm, energies_hbm, se_hbm, en_out_hbm,
          w_v, tb_v, se_v, en_v, out_v):
        wid = lax.axis_index("s") * info.num_cores + lax.axis_index("c")
        rbase = wid * rows_w
        pltpu.sync_copy(se_hbm, se_v)
        pltpu.sync_copy(energies_hbm.at[pl.ds(rbase, rows_w)], en_v)
        pltpu.sync_copy(w_hbm.at[pl.ds(rbase * _W, rows_w * _W)], w_v)

        lanes = lax.iota(jnp.int32, _L)

        def build(i, carry):
            e = i * _L + lanes
            t = (plsc.load_gather(se_v, [e & 7])
                 + plsc.load_gather(se_v, [lax.shift_right_logical(e, 8) & 7]))
            tb_v[pl.ds(i * _L, _L)] = t
            return carry

        lax.fori_loop(0, 2048 // _L, build, 0)

        def group(g, carry):
            rowflat = (g * _L + lanes) * _W
            accs = [jnp.zeros((_L,), jnp.float32) for _ in range(2)]

            def step(j5, accs):
                a0, a1 = accs
                base = rowflat + j5 * 10
                for u in range(10):
                    gw = plsc.load_gather(w_v, [base + u])
                    lo = gw & 0x7FF
                    hi = lax.shift_right_logical(gw, 16) & 0x7FF
                    a0 = a0 + plsc.load_gather(tb_v, [lo])
                    a1 = a1 + plsc.load_gather(tb_v, [hi])
                return (a0, a1)

            accs = lax.fori_loop(0, _W // 10, step, tuple(accs))
            o = g * _L
            out_v[pl.ds(o, _L)] = (accs[0] + accs[1]) + en_v[pl.ds(o, _L)]
            return carry

        lax.fori_loop(0, groups, group, 0)
        pltpu.sync_copy(out_v, en_out_hbm.at[pl.ds(rbase, rows_w)])

    return k(wpacked, energies, se_pad)


def kernel(species, energies, self_energies):
    wpacked = lax.bitcast_convert_type(
        species.astype(jnp.int8).reshape(_NROWS, _W, 4), jnp.int32
    ).reshape(-1)
    se_pad = jnp.concatenate(
        [self_energies.astype(jnp.float32),
         jnp.zeros((_L - self_energies.shape[0],), jnp.float32)]
    )
    new_energies = _sc_energy_shift(
        wpacked, energies.astype(jnp.float32), se_pad)
    return (species, new_energies)


# speedup vs baseline: 1.5925x; 1.5925x over previous
import functools

import jax
import jax.numpy as jnp
from jax import lax
from jax.experimental import pallas as pl
from jax.experimental.pallas import tpu as pltpu
from jax.experimental.pallas import tpu_sc as plsc

_NROWS = 16384
_NCOLS = 200
_P = 4
_WR = _NROWS // _P
_L = 16
_FULL = (_NCOLS // _L) * _L
_TAIL = _NCOLS - _FULL


def _sc_energy_shift(wpacked, energies, se_pad):
    info = plsc.get_sparse_core_info()
    nw = info.num_cores * info.num_subcores
    rows_w = _WR // nw

    mesh = plsc.VectorSubcoreMesh(core_axis_name="c", subcore_axis_name="s")

    @functools.partial(
        pl.kernel,
        mesh=mesh,
        out_type=jax.ShapeDtypeStruct((_NROWS,), jnp.float32),
        compiler_params=pltpu.CompilerParams(
            needs_layout_passes=False, use_tc_tiling_on_sc=False),
        scratch_types=[
            pltpu.VMEM((rows_w, _NCOLS), jnp.int32),
            pltpu.VMEM((_L,), jnp.float32),
            pltpu.VMEM((_L * _L,), jnp.float32),
            pltpu.VMEM((_P * rows_w,), jnp.float32),
            pltpu.VMEM((_P * rows_w,), jnp.float32),
        ],
    )
    def k(w_hbm, energies_hbm, se_hbm, en_out_hbm,
          w_v, se_v, serep_v, en_v, out_v):
        wid = lax.axis_index("s") * info.num_cores + lax.axis_index("c")
        rbase = wid * rows_w
        pltpu.sync_copy(se_hbm, se_v)
        pltpu.sync_copy(w_hbm.at[pl.ds(rbase, rows_w), :], w_v)
        for p in range(_P):
            pltpu.sync_copy(
                energies_hbm.at[pl.ds(p * _WR + rbase, rows_w)],
                en_v.at[pl.ds(p * rows_w, rows_w)])

        lanes = lax.iota(jnp.int32, _L)
        zeros = jnp.zeros((_L,), jnp.float32)
        se16 = se_v[...]
        for v in range(_L):
            plsc.store_scatter(serep_v, [lanes * _L + v], se16)
        tailsel = jnp.where(lanes < _TAIL, 8 * _L, 0).astype(jnp.int32)
        lane0 = lanes == 0

        def lookup(w16, accs, force):
            w4 = lax.shift_left(w16, 4)
            out = []
            for p in range(_P):
                a = (lax.shift_right_logical(w4, 8 * p) & 0xF0) + lanes
                if force is not None:
                    a = a | force
                out.append(accs[p] + plsc.load_gather(serep_v, [a]))
            return out

        def two_rows(i, carry):
            for rr in range(2):
                r = i * 2 + rr
                accs = [zeros for _ in range(_P)]
                for c in range(0, _FULL, _L):
                    accs = lookup(w_v[r, pl.ds(c, _L)], accs, None)
                accs = lookup(w_v[r, pl.ds(_NCOLS - _L, _L)], accs, tailsel)
                rvec = jnp.zeros((_L,), jnp.int32) + r
                for p in range(_P):
                    plsc.store_scatter(
                        out_v, [rvec + (p * rows_w)],
                        zeros + jnp.sum(accs[p]), mask=lane0)
            return carry

        lax.fori_loop(0, rows_w // 2, two_rows, 0)

        def add_en(i, carry):
            sl = pl.ds(i * _L, _L)
            out_v[sl] = out_v[sl] + en_v[sl]
            return carry

        lax.fori_loop(0, _P * rows_w // _L, add_en, 0)
        for p in range(_P):
            pltpu.sync_copy(
                out_v.at[pl.ds(p * rows_w, rows_w)],
                en_out_hbm.at[pl.ds(p * _WR + rbase, rows_w)])

    return k(wpacked, energies, se_pad)


def kernel(species, energies, self_energies):
    sp = species.astype(jnp.int32).reshape(_P, _WR, _NCOLS)
    w = (sp[0] | lax.shift_left(sp[1], 8) | lax.shift_left(sp[2], 16)
         | lax.shift_left(sp[3], 24))
    se_pad = jnp.concatenate(
        [self_energies.astype(jnp.float32),
         jnp.zeros((_L - self_energies.shape[0],), jnp.float32)]
    )
    new_energies = _sc_energy_shift(w, energies.astype(jnp.float32), se_pad)
    return (species, new_energies)


# speedup vs baseline: 1.8463x; 1.1594x over previous
import functools

import jax
import jax.numpy as jnp
from jax import lax
from jax.experimental import pallas as pl
from jax.experimental.pallas import tpu as pltpu
from jax.experimental.pallas import tpu_sc as plsc

_NROWS = 16384
_NCOLS = 200
_L = 16
_BANDS = _NCOLS // 8


def _sc_energy_shift(spT, energies, se_pad):
    info = plsc.get_sparse_core_info()
    nw = info.num_cores * info.num_subcores
    rows_w = _NROWS // nw
    groups = rows_w // _L

    mesh = plsc.VectorSubcoreMesh(core_axis_name="c", subcore_axis_name="s")

    @functools.partial(
        pl.kernel,
        mesh=mesh,
        out_type=jax.ShapeDtypeStruct((_NROWS,), jnp.float32),
        compiler_params=pltpu.CompilerParams(needs_layout_passes=False),
        scratch_types=[
            pltpu.VMEM((_BANDS, 8, rows_w), jnp.int32),
            pltpu.VMEM((_L,), jnp.float32),
            pltpu.VMEM((_L * _L,), jnp.float32),
            pltpu.VMEM((rows_w,), jnp.float32),
            pltpu.VMEM((rows_w,), jnp.float32),
        ],
    )
    def k(spT_hbm, energies_hbm, se_hbm, en_out_hbm,
          sp_v, se_v, serep_v, en_v, out_v):
        wid = lax.axis_index("s") * info.num_cores + lax.axis_index("c")
        rbase = wid * rows_w
        pltpu.sync_copy(se_hbm, se_v)
        pltpu.sync_copy(energies_hbm.at[pl.ds(rbase, rows_w)], en_v)
        for b in range(_BANDS):
            pltpu.sync_copy(
                spT_hbm.at[pl.ds(b * 8, 8), pl.ds(rbase, rows_w)],
                sp_v.at[b])

        lanes = lax.iota(jnp.int32, _L)
        se16 = se_v[...]
        for v in range(_L):
            plsc.store_scatter(serep_v, [lanes * _L + v], se16)

        def group(g, carry):
            r0 = g * _L
            accs = [jnp.zeros((_L,), jnp.float32) for _ in range(4)]

            def band(b, accs):
                accs = list(accs)
                for c in range(8):
                    s = sp_v[b, c, pl.ds(r0, _L)]
                    a = lax.shift_left(s, 4) + lanes
                    accs[c % 4] = accs[c % 4] + plsc.load_gather(serep_v, [a])
                return tuple(accs)

            accs = lax.fori_loop(0, _BANDS, band, tuple(accs))
            acc = (accs[0] + accs[1]) + (accs[2] + accs[3])
            out_v[pl.ds(r0, _L)] = acc + en_v[pl.ds(r0, _L)]
            return carry

        lax.fori_loop(0, groups, group, 0)
        pltpu.sync_copy(out_v, en_out_hbm.at[pl.ds(rbase, rows_w)])

    return k(spT, energies, se_pad)


def kernel(species, energies, self_energies):
    spT = species.astype(jnp.int32).T
    se_pad = jnp.concatenate(
        [self_energies.astype(jnp.float32),
         jnp.zeros((_L - self_energies.shape[0],), jnp.float32)]
    )
    new_energies = _sc_energy_shift(spT, energies.astype(jnp.float32), se_pad)
    return (species, new_energies)


# speedup vs baseline: 3.1670x; 1.7153x over previous
import functools

import jax
import jax.numpy as jnp
from jax import lax
from jax.experimental import pallas as pl
from jax.experimental.pallas import tpu as pltpu
from jax.experimental.pallas import tpu_sc as plsc

_NROWS = 16384
_NCOLS = 200
_L = 16


def _sc_energy_shift(spT, energies, se_pad):
    info = plsc.get_sparse_core_info()
    nw = info.num_cores * info.num_subcores
    rows_w = _NROWS // nw
    groups = rows_w // _L

    mesh = plsc.VectorSubcoreMesh(core_axis_name="c", subcore_axis_name="s")

    @functools.partial(
        pl.kernel,
        mesh=mesh,
        out_type=(
            jax.ShapeDtypeStruct((_NCOLS, _NROWS), jnp.int32),
            jax.ShapeDtypeStruct((_NROWS,), jnp.float32),
        ),
        compiler_params=pltpu.CompilerParams(needs_layout_passes=False),
        scratch_types=[
            pltpu.VMEM((_NCOLS, rows_w), jnp.int32),
            pltpu.VMEM((_L,), jnp.float32),
            pltpu.VMEM((_L * _L,), jnp.float32),
            pltpu.VMEM((rows_w,), jnp.float32),
            pltpu.VMEM((rows_w,), jnp.float32),
            pltpu.SemaphoreType.DMA,
        ],
    )
    def k(spT_hbm, energies_hbm, se_hbm, spT_out_hbm, en_out_hbm,
          sp_v, se_v, serep_v, en_v, out_v, sem):
        wid = lax.axis_index("s") * info.num_cores + lax.axis_index("c")
        rbase = wid * rows_w
        pltpu.sync_copy(se_hbm, se_v)
        pltpu.sync_copy(energies_hbm.at[pl.ds(rbase, rows_w)], en_v)
        pltpu.sync_copy(spT_hbm.at[:, pl.ds(rbase, rows_w)], sp_v)
        out_dma = pltpu.async_copy(
            sp_v, spT_out_hbm.at[:, pl.ds(rbase, rows_w)], sem)

        lanes = lax.iota(jnp.int32, _L)
        se16 = se_v[...]
        for v in range(_L):
            plsc.store_scatter(serep_v, [lanes * _L + v], se16)

        def group(g, carry):
            r0 = g * _L
            accs = [jnp.zeros((_L,), jnp.float32) for _ in range(4)]

            def band(b, accs):
                accs = list(accs)
                for c in range(8):
                    s = sp_v[b * 8 + c, pl.ds(r0, _L)]
                    a = lax.shift_left(s, 4) + lanes
                    accs[c % 4] = accs[c % 4] + plsc.load_gather(serep_v, [a])
                return tuple(accs)

            accs = lax.fori_loop(0, _NCOLS // 8, band, tuple(accs))
            acc = (accs[0] + accs[1]) + (accs[2] + accs[3])
            out_v[pl.ds(r0, _L)] = acc + en_v[pl.ds(r0, _L)]
            return carry

        lax.fori_loop(0, groups, group, 0)
        pltpu.sync_copy(out_v, en_out_hbm.at[pl.ds(rbase, rows_w)])
        out_dma.wait()

    return k(spT, energies, se_pad)


def kernel(species, energies, self_energies):
    spT = species.astype(jnp.int32).T
    se_pad = jnp.concatenate(
        [self_energies.astype(jnp.float32),
         jnp.zeros((_L - self_energies.shape[0],), jnp.float32)]
    )
    spT_out, new_energies = _sc_energy_shift(
        spT, energies.astype(jnp.float32), se_pad)
    return (spT_out.T.astype(species.dtype), new_energies)


# speedup vs baseline: 3.1695x; 1.0008x over previous
import functools

import jax
import jax.numpy as jnp
from jax import lax
from jax.experimental import pallas as pl
from jax.experimental.pallas import tpu as pltpu
from jax.experimental.pallas import tpu_sc as plsc

_NROWS = 16384
_NCOLS = 200
_L = 16


def _sc_energy_shift(spT, energies, self_energies):
    info = plsc.get_sparse_core_info()
    nw = info.num_cores * info.num_subcores
    rows_w = _NROWS // nw
    half = rows_w // 2
    hgroups = half // _L

    mesh = plsc.VectorSubcoreMesh(core_axis_name="c", subcore_axis_name="s")

    @functools.partial(
        pl.kernel,
        mesh=mesh,
        out_type=(
            jax.ShapeDtypeStruct((_NCOLS, _NROWS), jnp.int32),
            jax.ShapeDtypeStruct((_NROWS,), jnp.float32),
        ),
        compiler_params=pltpu.CompilerParams(needs_layout_passes=False),
        scratch_types=[
            pltpu.VMEM((_NCOLS, rows_w), jnp.int32),
            pltpu.VMEM((_L,), jnp.float32),
            pltpu.VMEM((_L * _L,), jnp.float32),
            pltpu.VMEM((rows_w,), jnp.float32),
            pltpu.VMEM((rows_w,), jnp.float32),
            pltpu.SemaphoreType.DMA,
            pltpu.SemaphoreType.DMA,
            pltpu.SemaphoreType.DMA,
            pltpu.SemaphoreType.DMA,
        ],
    )
    def k(spT_hbm, energies_hbm, se_hbm, spT_out_hbm, en_out_hbm,
          sp_v, se_v, serep_v, en_v, out_v, si0, si1, so0, so1):
        wid = lax.axis_index("s") * info.num_cores + lax.axis_index("c")
        rbase = wid * rows_w
        lanes = lax.iota(jnp.int32, _L)
        zeros = jnp.zeros((_L,), jnp.float32)

        ins = [
            pltpu.async_copy(
                spT_hbm.at[:, pl.ds(rbase + h * half, half)],
                sp_v.at[:, pl.ds(h * half, half)], sem)
            for h, sem in ((0, si0), (1, si1))
        ]
        se_v[...] = zeros
        pltpu.sync_copy(se_hbm, se_v.at[pl.ds(0, 8)])
        pltpu.sync_copy(energies_hbm.at[pl.ds(rbase, rows_w)], en_v)

        se16 = se_v[...]
        for v in range(_L):
            plsc.store_scatter(serep_v, [lanes * _L + v], se16)

        def group(g, carry):
            r0 = g * _L
            accs = [zeros for _ in range(4)]

            def chunk(b, accs):
                accs = list(accs)
                for u in range(40):
                    s = sp_v[b * 40 + u, pl.ds(r0, _L)]
                    a = lax.shift_left(s, 4) + lanes
                    accs[u % 4] = accs[u % 4] + plsc.load_gather(serep_v, [a])
                return tuple(accs)

            accs = lax.fori_loop(0, _NCOLS // 40, chunk, tuple(accs))
            acc = (accs[0] + accs[1]) + (accs[2] + accs[3])
            out_v[pl.ds(r0, _L)] = acc + en_v[pl.ds(r0, _L)]
            return carry

        outs = []
        for h, sem in ((0, so0), (1, so1)):
            ins[h].wait()
            outs.append(pltpu.async_copy(
                sp_v.at[:, pl.ds(h * half, half)],
                spT_out_hbm.at[:, pl.ds(rbase + h * half, half)], sem))
            lax.fori_loop(h * hgroups, (h + 1) * hgroups, group, 0)

        pltpu.sync_copy(out_v, en_out_hbm.at[pl.ds(rbase, rows_w)])
        for o in outs:
            o.wait()

    return k(spT, energies, self_energies)


def kernel(species, energies, self_energies):
    spT = species.astype(jnp.int32).T
    spT_out, new_energies = _sc_energy_shift(
        spT, energies.astype(jnp.float32), self_energies.astype(jnp.float32))
    return (spT_out.T.astype(species.dtype), new_energies)
